# Initial kernel scaffold; baseline (speedup 1.0000x reference)
import jax
import jax.numpy as jnp
from jax.experimental import pallas as pl


def kernel(feat, feat_a, feat_b, adj, graph_neigh, W1, W2, dec_W, dec_b, bn_gamma, bn_beta, pi_W, pi_b, disp_W, disp_b, mean_W, mean_b, bil_W, bil_b):
    raise NotImplementedError("write your pallas kernel here")


# speedup vs baseline: 1.3560x; 1.3560x over previous
import jax
import jax.numpy as jnp
from jax.experimental import pallas as pl

INTERPRET = False

_BI_B = 256
_BI_C = 256
_BI_D = 192
_BI_F = 256

_BF = jnp.bfloat16
_F32 = jnp.float32


def _bdot(a, b):
    return jax.lax.dot(a.astype(_BF), b.astype(_BF),
                       preferred_element_type=_F32)


def _proj_kernel(x_ref, w1_ref, p_ref):
    p_ref[0] = _bdot(x_ref[0], w1_ref[...]).astype(_BF)


def _layer1_kernel(adj_ref, p_ref, w2_ref, q_ref):
    a = adj_ref[...].astype(_BF)
    w2 = w2_ref[...].astype(_BF)
    for g in range(3):
        h = jax.lax.dot(a, p_ref[g], preferred_element_type=_F32)
        h = jnp.maximum(h, 0.0)
        q_ref[g] = jax.lax.dot(h.astype(_BF), w2,
                               preferred_element_type=_F32).astype(_BF)


def _layer2_kernel(adj_ref, q_ref, z_ref, zn_ref):
    a = adj_ref[...].astype(_BF)
    z0 = None
    for g in range(3):
        zg = jax.lax.dot(a, q_ref[g], preferred_element_type=_F32)
        z_ref[g] = zg
        if g == 0:
            z0 = zg
    n = jnp.sqrt(jnp.sum(z0 * z0, axis=1, keepdims=True))
    zn_ref[...] = z0 / jnp.maximum(n, 1e-12)


def _recon_kernel(gn_ref, z_ref, znb_ref, znf_ref, rec_ref, g2_ref):
    gn = gn_ref[...]
    emb1 = jnp.maximum(z_ref[0], 0.0)
    vs = _bdot(gn, emb1)
    rs = jnp.sum(gn, axis=1, keepdims=True)
    v = vs / rs
    nv = jnp.sqrt(jnp.sum(v * v, axis=1, keepdims=True))
    g2_ref[...] = jax.nn.sigmoid(v / jnp.maximum(nv, 1e-12))
    rec = jax.lax.dot_general(
        znb_ref[...].astype(_BF), znf_ref[...].astype(_BF),
        (((1,), (1,)), ((), ())), preferred_element_type=_F32)
    rec_ref[...] = jax.nn.sigmoid(rec)


def _bn_kernel(z_ref, dw_ref, db_ref, bg_ref, bb_ref, xr_ref):
    xd = _bdot(z_ref[0], dw_ref[...]) + db_ref[...]
    mu = jnp.mean(xd, axis=0, keepdims=True)
    var = jnp.mean((xd - mu) ** 2, axis=0, keepdims=True)
    xn = (xd - mu) / jnp.sqrt(var + 1e-5) * bg_ref[...] + bb_ref[...]
    xr_ref[...] = jnp.maximum(xn, 0.0)


def _heads_kernel(xr_ref, z_ref, g2_ref, piw_ref, pib_ref, dw_ref, db_ref,
                  mw_ref, mb_ref, bw_ref, bb_ref,
                  pi_ref, disp_ref, mean_ref, ret_ref):
    xr = xr_ref[...]
    pi_ref[...] = jax.nn.sigmoid(_bdot(xr, piw_ref[...]) + pib_ref[...])
    disp_ref[...] = jnp.clip(
        jax.nn.softplus(_bdot(xr, dw_ref[...]) + db_ref[...]), 1e-4, 1e4)
    mean_ref[...] = jnp.clip(
        jnp.exp(_bdot(xr, mw_ref[...]) + mb_ref[...]), 1e-5, 1e6)
    g2 = g2_ref[...]
    emb1 = jnp.maximum(z_ref[0], 0.0)
    emb3 = jnp.maximum(z_ref[2], 0.0)
    t1 = _bdot(emb1, bw_ref[...])
    t3 = _bdot(emb3, bw_ref[...])
    sc1 = jnp.sum(t1 * g2, axis=1, keepdims=True) + bb_ref[...]
    sc2 = jnp.sum(t3 * g2, axis=1, keepdims=True) + bb_ref[...]
    ret_ref[:, 0:1] = sc1
    ret_ref[:, 1:2] = sc2


def kernel(feat, feat_a, feat_b, adj, graph_neigh, W1, W2, dec_W, dec_b,
           bn_gamma, bn_beta, pi_W, pi_b, disp_W, disp_b, mean_W, mean_b,
           bil_W, bil_b):
    n, d_in = feat.shape
    h1 = W1.shape[1]
    h2 = W2.shape[1]

    xs = jnp.stack([feat, feat_a, feat_b], axis=0)

    p = pl.pallas_call(
        _proj_kernel,
        grid=(3,),
        in_specs=[
            pl.BlockSpec((1, n, d_in), lambda g: (g, 0, 0)),
            pl.BlockSpec((d_in, h1), lambda g: (0, 0)),
        ],
        out_specs=pl.BlockSpec((1, n, h1), lambda g: (g, 0, 0)),
        out_shape=jax.ShapeDtypeStruct((3, n, h1), _BF),
        interpret=INTERPRET,
    )(xs, W1)

    q = pl.pallas_call(
        _layer1_kernel,
        grid=(pl.cdiv(n, _BI_B),),
        in_specs=[
            pl.BlockSpec((_BI_B, n), lambda i: (i, 0)),
            pl.BlockSpec((3, n, h1), lambda i: (0, 0, 0)),
            pl.BlockSpec((h1, h2), lambda i: (0, 0)),
        ],
        out_specs=pl.BlockSpec((3, _BI_B, h2), lambda i: (0, i, 0)),
        out_shape=jax.ShapeDtypeStruct((3, n, h2), _BF),
        interpret=INTERPRET,
    )(adj, p, W2)

    z, zn = pl.pallas_call(
        _layer2_kernel,
        grid=(pl.cdiv(n, _BI_C),),
        in_specs=[
            pl.BlockSpec((_BI_C, n), lambda i: (i, 0)),
            pl.BlockSpec((3, n, h2), lambda i: (0, 0, 0)),
        ],
        out_specs=[
            pl.BlockSpec((3, _BI_C, h2), lambda i: (0, i, 0)),
            pl.BlockSpec((_BI_C, h2), lambda i: (i, 0)),
        ],
        out_shape=[
            jax.ShapeDtypeStruct((3, n, h2), _F32),
            jax.ShapeDtypeStruct((n, h2), _F32),
        ],
        interpret=INTERPRET,
    )(adj, q)

    rec_adj, g2 = pl.pallas_call(
        _recon_kernel,
        grid=(pl.cdiv(n, _BI_D),),
        in_specs=[
            pl.BlockSpec((_BI_D, n), lambda i: (i, 0)),
            pl.BlockSpec((3, n, h2), lambda i: (0, 0, 0)),
            pl.BlockSpec((_BI_D, h2), lambda i: (i, 0)),
            pl.BlockSpec((n, h2), lambda i: (0, 0)),
        ],
        out_specs=[
            pl.BlockSpec((_BI_D, n), lambda i: (i, 0)),
            pl.BlockSpec((_BI_D, h2), lambda i: (i, 0)),
        ],
        out_shape=[
            jax.ShapeDtypeStruct((n, n), _F32),
            jax.ShapeDtypeStruct((n, h2), _F32),
        ],
        interpret=INTERPRET,
    )(graph_neigh, z, zn, zn)

    xr = pl.pallas_call(
        _bn_kernel,
        grid=(1,),
        in_specs=[
            pl.BlockSpec((3, n, h2), lambda i: (0, 0, 0)),
            pl.BlockSpec((h2, h1), lambda i: (0, 0)),
            pl.BlockSpec((1, h1), lambda i: (0, 0)),
            pl.BlockSpec((1, h1), lambda i: (0, 0)),
            pl.BlockSpec((1, h1), lambda i: (0, 0)),
        ],
        out_specs=pl.BlockSpec((n, h1), lambda i: (0, 0)),
        out_shape=jax.ShapeDtypeStruct((n, h1), _F32),
        interpret=INTERPRET,
    )(z, dec_W, dec_b.reshape(1, h1), bn_gamma.reshape(1, h1),
      bn_beta.reshape(1, h1))

    pi, disp, mean_, ret1 = pl.pallas_call(
        _heads_kernel,
        grid=(pl.cdiv(n, _BI_F),),
        in_specs=[
            pl.BlockSpec((_BI_F, h1), lambda i: (i, 0)),
            pl.BlockSpec((3, _BI_F, h2), lambda i: (0, i, 0)),
            pl.BlockSpec((_BI_F, h2), lambda i: (i, 0)),
            pl.BlockSpec((h1, d_in), lambda i: (0, 0)),
            pl.BlockSpec((1, d_in), lambda i: (0, 0)),
            pl.BlockSpec((h1, d_in), lambda i: (0, 0)),
            pl.BlockSpec((1, d_in), lambda i: (0, 0)),
            pl.BlockSpec((h1, d_in), lambda i: (0, 0)),
            pl.BlockSpec((1, d_in), lambda i: (0, 0)),
            pl.BlockSpec((h2, h2), lambda i: (0, 0)),
            pl.BlockSpec((1, 1), lambda i: (0, 0)),
        ],
        out_specs=[
            pl.BlockSpec((_BI_F, d_in), lambda i: (i, 0)),
            pl.BlockSpec((_BI_F, d_in), lambda i: (i, 0)),
            pl.BlockSpec((_BI_F, d_in), lambda i: (i, 0)),
            pl.BlockSpec((_BI_F, 2), lambda i: (i, 0)),
        ],
        out_shape=[
            jax.ShapeDtypeStruct((n, d_in), _F32),
            jax.ShapeDtypeStruct((n, d_in), _F32),
            jax.ShapeDtypeStruct((n, d_in), _F32),
            jax.ShapeDtypeStruct((n, 2), _F32),
        ],
        interpret=INTERPRET,
    )(xr, z, g2, pi_W, pi_b.reshape(1, d_in), disp_W, disp_b.reshape(1, d_in),
      mean_W, mean_b.reshape(1, d_in), bil_W, bil_b.reshape(1, 1))

    return (z[0], z[1], z[2], pi, disp, mean_, rec_adj, ret1)


# speedup vs baseline: 1.6702x; 1.2318x over previous
import jax
import jax.numpy as jnp
from jax.experimental import pallas as pl

INTERPRET = False

_BI_B = 256
_BI_C = 256
_BI_D = 256
_BI_F = 256

_BF = jnp.bfloat16
_F32 = jnp.float32


def _bdot(a, b):
    return jax.lax.dot(a.astype(_BF), b.astype(_BF),
                       preferred_element_type=_F32)


def _proj_kernel(x_ref, w1_ref, p_ref):
    p_ref[...] = _bdot(x_ref[0], w1_ref[...]).astype(_BF)


def _layer1_kernel(adj_ref, p_ref, w2d_ref, q_ref):
    a = adj_ref[...].astype(_BF)
    h = jax.lax.dot(a, p_ref[...], preferred_element_type=_F32)
    h = jnp.maximum(h, 0.0)
    q_ref[...] = jax.lax.dot(h.astype(_BF), w2d_ref[...],
                             preferred_element_type=_F32).astype(_BF)


def _layer2_kernel(adj_ref, q_ref, z_ref, zn_ref, e1_ref):
    a = adj_ref[...].astype(_BF)
    zall = jax.lax.dot(a, q_ref[...], preferred_element_type=_F32)
    h2 = zn_ref.shape[-1]
    for g in range(3):
        z_ref[g] = zall[:, g * h2:(g + 1) * h2]
    z0 = zall[:, :h2]
    n = jnp.sqrt(jnp.sum(z0 * z0, axis=1, keepdims=True))
    zn_ref[...] = z0 / jnp.maximum(n, 1e-12)
    e1_ref[...] = jnp.maximum(z0, 0.0)


def _recon_kernel(gn_ref, e1_ref, znb_ref, znf_ref, rec_ref, g2_ref):
    gn = gn_ref[...]
    vs = _bdot(gn, e1_ref[...])
    rs = jnp.sum(gn, axis=1, keepdims=True)
    v = vs / rs
    nv = jnp.sqrt(jnp.sum(v * v, axis=1, keepdims=True))
    g2_ref[...] = jax.nn.sigmoid(v / jnp.maximum(nv, 1e-12))
    rec = jax.lax.dot_general(
        znb_ref[...].astype(_BF), znf_ref[...].astype(_BF),
        (((1,), (1,)), ((), ())), preferred_element_type=_F32)
    rec_ref[...] = jax.nn.sigmoid(rec)


def _bn_kernel(z_ref, dw_ref, db_ref, bg_ref, bb_ref, xr_ref):
    xd = _bdot(z_ref[0], dw_ref[...]) + db_ref[...]
    mu = jnp.mean(xd, axis=0, keepdims=True)
    var = jnp.mean((xd - mu) ** 2, axis=0, keepdims=True)
    xn = (xd - mu) / jnp.sqrt(var + 1e-5) * bg_ref[...] + bb_ref[...]
    xr_ref[...] = jnp.maximum(xn, 0.0)


def _heads_kernel(xr_ref, z_ref, g2_ref, piw_ref, pib_ref, dw_ref, db_ref,
                  mw_ref, mb_ref, bw_ref, bb_ref,
                  pi_ref, disp_ref, mean_ref, ret_ref):
    xr = xr_ref[...]
    pi_ref[...] = jax.nn.sigmoid(_bdot(xr, piw_ref[...]) + pib_ref[...])
    disp_ref[...] = jnp.clip(
        jax.nn.softplus(_bdot(xr, dw_ref[...]) + db_ref[...]), 1e-4, 1e4)
    mean_ref[...] = jnp.clip(
        jnp.exp(_bdot(xr, mw_ref[...]) + mb_ref[...]), 1e-5, 1e6)
    g2 = g2_ref[...]
    emb1 = jnp.maximum(z_ref[0], 0.0)
    emb3 = jnp.maximum(z_ref[2], 0.0)
    t1 = _bdot(emb1, bw_ref[...])
    t3 = _bdot(emb3, bw_ref[...])
    sc1 = jnp.sum(t1 * g2, axis=1, keepdims=True) + bb_ref[...]
    sc2 = jnp.sum(t3 * g2, axis=1, keepdims=True) + bb_ref[...]
    ret_ref[:, 0:1] = sc1
    ret_ref[:, 1:2] = sc2


def kernel(feat, feat_a, feat_b, adj, graph_neigh, W1, W2, dec_W, dec_b,
           bn_gamma, bn_beta, pi_W, pi_b, disp_W, disp_b, mean_W, mean_b,
           bil_W, bil_b):
    n, d_in = feat.shape
    h1 = W1.shape[1]
    h2 = W2.shape[1]

    xs = jnp.stack([feat, feat_a, feat_b], axis=0)
    zero = jnp.zeros_like(W2)
    w2d = jnp.block([[W2, zero, zero],
                     [zero, W2, zero],
                     [zero, zero, W2]]).astype(_BF)

    p = pl.pallas_call(
        _proj_kernel,
        grid=(3,),
        in_specs=[
            pl.BlockSpec((1, n, d_in), lambda g: (g, 0, 0)),
            pl.BlockSpec((d_in, h1), lambda g: (0, 0)),
        ],
        out_specs=pl.BlockSpec((n, h1), lambda g: (0, g)),
        out_shape=jax.ShapeDtypeStruct((n, 3 * h1), _BF),
        interpret=INTERPRET,
    )(xs, W1)

    q = pl.pallas_call(
        _layer1_kernel,
        grid=(pl.cdiv(n, _BI_B),),
        in_specs=[
            pl.BlockSpec((_BI_B, n), lambda i: (i, 0)),
            pl.BlockSpec((n, 3 * h1), lambda i: (0, 0)),
            pl.BlockSpec((3 * h1, 3 * h2), lambda i: (0, 0)),
        ],
        out_specs=pl.BlockSpec((_BI_B, 3 * h2), lambda i: (i, 0)),
        out_shape=jax.ShapeDtypeStruct((n, 3 * h2), _BF),
        interpret=INTERPRET,
    )(adj, p, w2d)

    z, zn, emb1 = pl.pallas_call(
        _layer2_kernel,
        grid=(pl.cdiv(n, _BI_C),),
        in_specs=[
            pl.BlockSpec((_BI_C, n), lambda i: (i, 0)),
            pl.BlockSpec((n, 3 * h2), lambda i: (0, 0)),
        ],
        out_specs=[
            pl.BlockSpec((3, _BI_C, h2), lambda i: (0, i, 0)),
            pl.BlockSpec((_BI_C, h2), lambda i: (i, 0)),
            pl.BlockSpec((_BI_C, h2), lambda i: (i, 0)),
        ],
        out_shape=[
            jax.ShapeDtypeStruct((3, n, h2), _F32),
            jax.ShapeDtypeStruct((n, h2), _F32),
            jax.ShapeDtypeStruct((n, h2), _F32),
        ],
        interpret=INTERPRET,
    )(adj, q)

    rec_adj, g2 = pl.pallas_call(
        _recon_kernel,
        grid=(pl.cdiv(n, _BI_D),),
        in_specs=[
            pl.BlockSpec((_BI_D, n), lambda i: (i, 0)),
            pl.BlockSpec((n, h2), lambda i: (0, 0)),
            pl.BlockSpec((_BI_D, h2), lambda i: (i, 0)),
            pl.BlockSpec((n, h2), lambda i: (0, 0)),
        ],
        out_specs=[
            pl.BlockSpec((_BI_D, n), lambda i: (i, 0)),
            pl.BlockSpec((_BI_D, h2), lambda i: (i, 0)),
        ],
        out_shape=[
            jax.ShapeDtypeStruct((n, n), _F32),
            jax.ShapeDtypeStruct((n, h2), _F32),
        ],
        interpret=INTERPRET,
    )(graph_neigh, emb1, zn, zn)

    xr = pl.pallas_call(
        _bn_kernel,
        grid=(1,),
        in_specs=[
            pl.BlockSpec((3, n, h2), lambda i: (0, 0, 0)),
            pl.BlockSpec((h2, h1), lambda i: (0, 0)),
            pl.BlockSpec((1, h1), lambda i: (0, 0)),
            pl.BlockSpec((1, h1), lambda i: (0, 0)),
            pl.BlockSpec((1, h1), lambda i: (0, 0)),
        ],
        out_specs=pl.BlockSpec((n, h1), lambda i: (0, 0)),
        out_shape=jax.ShapeDtypeStruct((n, h1), _F32),
        interpret=INTERPRET,
    )(z, dec_W, dec_b.reshape(1, h1), bn_gamma.reshape(1, h1),
      bn_beta.reshape(1, h1))

    pi, disp, mean_, ret1 = pl.pallas_call(
        _heads_kernel,
        grid=(pl.cdiv(n, _BI_F),),
        in_specs=[
            pl.BlockSpec((_BI_F, h1), lambda i: (i, 0)),
            pl.BlockSpec((3, _BI_F, h2), lambda i: (0, i, 0)),
            pl.BlockSpec((_BI_F, h2), lambda i: (i, 0)),
            pl.BlockSpec((h1, d_in), lambda i: (0, 0)),
            pl.BlockSpec((1, d_in), lambda i: (0, 0)),
            pl.BlockSpec((h1, d_in), lambda i: (0, 0)),
            pl.BlockSpec((1, d_in), lambda i: (0, 0)),
            pl.BlockSpec((h1, d_in), lambda i: (0, 0)),
            pl.BlockSpec((1, d_in), lambda i: (0, 0)),
            pl.BlockSpec((h2, h2), lambda i: (0, 0)),
            pl.BlockSpec((1, 1), lambda i: (0, 0)),
        ],
        out_specs=[
            pl.BlockSpec((_BI_F, d_in), lambda i: (i, 0)),
            pl.BlockSpec((_BI_F, d_in), lambda i: (i, 0)),
            pl.BlockSpec((_BI_F, d_in), lambda i: (i, 0)),
            pl.BlockSpec((_BI_F, 2), lambda i: (i, 0)),
        ],
        out_shape=[
            jax.ShapeDtypeStruct((n, d_in), _F32),
            jax.ShapeDtypeStruct((n, d_in), _F32),
            jax.ShapeDtypeStruct((n, d_in), _F32),
            jax.ShapeDtypeStruct((n, 2), _F32),
        ],
        interpret=INTERPRET,
    )(xr, z, g2, pi_W, pi_b.reshape(1, d_in), disp_W, disp_b.reshape(1, d_in),
      mean_W, mean_b.reshape(1, d_in), bil_W, bil_b.reshape(1, 1))

    return (z[0], z[1], z[2], pi, disp, mean_, rec_adj, ret1)


# speedup vs baseline: 1.7505x; 1.0481x over previous
import jax
import jax.numpy as jnp
from jax.experimental import pallas as pl

INTERPRET = False

_BI_B = 256
_BI_C = 1000
_BI_D = 256
_BI_F = 2000

_BF = jnp.bfloat16
_F32 = jnp.float32


def _bdot(a, b):
    return jax.lax.dot(a.astype(_BF), b.astype(_BF),
                       preferred_element_type=_F32)


def _proj_kernel(x_ref, w1_ref, p_ref):
    p_ref[...] = _bdot(x_ref[0], w1_ref[...]).astype(_BF)


def _layer1_kernel(adj_ref, p_ref, w2d_ref, q_ref, abf_ref):
    a = adj_ref[...].astype(_BF)
    abf_ref[...] = a
    h = jax.lax.dot(a, p_ref[...], preferred_element_type=_F32)
    h = jnp.maximum(h, 0.0)
    q_ref[...] = jax.lax.dot(h.astype(_BF), w2d_ref[...],
                             preferred_element_type=_F32).astype(_BF)


def _layer2_kernel(adj_ref, q_ref, z_ref, zn_ref, e1_ref):
    a = adj_ref[...]
    zall = jax.lax.dot(a, q_ref[...], preferred_element_type=_F32)
    h2 = zn_ref.shape[-1]
    for g in range(3):
        z_ref[g] = zall[:, g * h2:(g + 1) * h2]
    z0 = zall[:, :h2]
    n = jnp.sqrt(jnp.sum(z0 * z0, axis=1, keepdims=True))
    zn_ref[...] = z0 / jnp.maximum(n, 1e-12)
    e1_ref[...] = jnp.maximum(z0, 0.0)


def _recon_kernel(gn_ref, e1_ref, znb_ref, znf_ref, rec_ref, g2_ref):
    vs = _bdot(gn_ref[...], e1_ref[...])
    nv = jnp.sqrt(jnp.sum(vs * vs, axis=1, keepdims=True))
    g2_ref[...] = jax.nn.sigmoid(vs / jnp.maximum(nv, 1e-12))
    rec = jax.lax.dot_general(
        znb_ref[...].astype(_BF), znf_ref[...].astype(_BF),
        (((1,), (1,)), ((), ())), preferred_element_type=_F32)
    rec_ref[...] = jax.nn.sigmoid(rec)


def _bn_kernel(z_ref, dw_ref, db_ref, bg_ref, bb_ref, xr_ref):
    xd = _bdot(z_ref[0], dw_ref[...]) + db_ref[...]
    mu = jnp.mean(xd, axis=0, keepdims=True)
    var = jnp.mean((xd - mu) ** 2, axis=0, keepdims=True)
    xn = (xd - mu) / jnp.sqrt(var + 1e-5) * bg_ref[...] + bb_ref[...]
    xr_ref[...] = jnp.maximum(xn, 0.0)


def _heads_kernel(xr_ref, z_ref, g2_ref, piw_ref, pib_ref, dw_ref, db_ref,
                  mw_ref, mb_ref, bw_ref, bb_ref,
                  pi_ref, disp_ref, mean_ref, ret_ref):
    xr = xr_ref[...]
    pi_ref[...] = jax.nn.sigmoid(_bdot(xr, piw_ref[...]) + pib_ref[...])
    disp_ref[...] = jnp.clip(
        jax.nn.softplus(_bdot(xr, dw_ref[...]) + db_ref[...]), 1e-4, 1e4)
    mean_ref[...] = jnp.clip(
        jnp.exp(_bdot(xr, mw_ref[...]) + mb_ref[...]), 1e-5, 1e6)
    g2 = g2_ref[...]
    emb1 = jnp.maximum(z_ref[0], 0.0)
    emb3 = jnp.maximum(z_ref[2], 0.0)
    t1 = _bdot(emb1, bw_ref[...])
    t3 = _bdot(emb3, bw_ref[...])
    sc1 = jnp.sum(t1 * g2, axis=1, keepdims=True) + bb_ref[...]
    sc2 = jnp.sum(t3 * g2, axis=1, keepdims=True) + bb_ref[...]
    ret_ref[:, 0:1] = sc1
    ret_ref[:, 1:2] = sc2


def kernel(feat, feat_a, feat_b, adj, graph_neigh, W1, W2, dec_W, dec_b,
           bn_gamma, bn_beta, pi_W, pi_b, disp_W, disp_b, mean_W, mean_b,
           bil_W, bil_b):
    n, d_in = feat.shape
    h1 = W1.shape[1]
    h2 = W2.shape[1]

    xs = jnp.stack([feat, feat_a, feat_b], axis=0)
    zero = jnp.zeros_like(W2)
    w2d = jnp.block([[W2, zero, zero],
                     [zero, W2, zero],
                     [zero, zero, W2]]).astype(_BF)

    p = pl.pallas_call(
        _proj_kernel,
        grid=(3,),
        in_specs=[
            pl.BlockSpec((1, n, d_in), lambda g: (g, 0, 0)),
            pl.BlockSpec((d_in, h1), lambda g: (0, 0)),
        ],
        out_specs=pl.BlockSpec((n, h1), lambda g: (0, g)),
        out_shape=jax.ShapeDtypeStruct((n, 3 * h1), _BF),
        interpret=INTERPRET,
    )(xs, W1)

    q, adj_bf = pl.pallas_call(
        _layer1_kernel,
        grid=(pl.cdiv(n, _BI_B),),
        in_specs=[
            pl.BlockSpec((_BI_B, n), lambda i: (i, 0)),
            pl.BlockSpec((n, 3 * h1), lambda i: (0, 0)),
            pl.BlockSpec((3 * h1, 3 * h2), lambda i: (0, 0)),
        ],
        out_specs=[
            pl.BlockSpec((_BI_B, 3 * h2), lambda i: (i, 0)),
            pl.BlockSpec((_BI_B, n), lambda i: (i, 0)),
        ],
        out_shape=[
            jax.ShapeDtypeStruct((n, 3 * h2), _BF),
            jax.ShapeDtypeStruct((n, n), _BF),
        ],
        interpret=INTERPRET,
    )(adj, p, w2d)

    z, zn, emb1 = pl.pallas_call(
        _layer2_kernel,
        grid=(pl.cdiv(n, _BI_C),),
        in_specs=[
            pl.BlockSpec((_BI_C, n), lambda i: (i, 0)),
            pl.BlockSpec((n, 3 * h2), lambda i: (0, 0)),
        ],
        out_specs=[
            pl.BlockSpec((3, _BI_C, h2), lambda i: (0, i, 0)),
            pl.BlockSpec((_BI_C, h2), lambda i: (i, 0)),
            pl.BlockSpec((_BI_C, h2), lambda i: (i, 0)),
        ],
        out_shape=[
            jax.ShapeDtypeStruct((3, n, h2), _F32),
            jax.ShapeDtypeStruct((n, h2), _F32),
            jax.ShapeDtypeStruct((n, h2), _F32),
        ],
        interpret=INTERPRET,
    )(adj_bf, q)

    rec_adj, g2 = pl.pallas_call(
        _recon_kernel,
        grid=(pl.cdiv(n, _BI_D),),
        in_specs=[
            pl.BlockSpec((_BI_D, n), lambda i: (i, 0)),
            pl.BlockSpec((n, h2), lambda i: (0, 0)),
            pl.BlockSpec((_BI_D, h2), lambda i: (i, 0)),
            pl.BlockSpec((n, h2), lambda i: (0, 0)),
        ],
        out_specs=[
            pl.BlockSpec((_BI_D, n), lambda i: (i, 0)),
            pl.BlockSpec((_BI_D, h2), lambda i: (i, 0)),
        ],
        out_shape=[
            jax.ShapeDtypeStruct((n, n), _F32),
            jax.ShapeDtypeStruct((n, h2), _F32),
        ],
        interpret=INTERPRET,
    )(graph_neigh, emb1, zn, zn)

    xr = pl.pallas_call(
        _bn_kernel,
        grid=(1,),
        in_specs=[
            pl.BlockSpec((3, n, h2), lambda i: (0, 0, 0)),
            pl.BlockSpec((h2, h1), lambda i: (0, 0)),
            pl.BlockSpec((1, h1), lambda i: (0, 0)),
            pl.BlockSpec((1, h1), lambda i: (0, 0)),
            pl.BlockSpec((1, h1), lambda i: (0, 0)),
        ],
        out_specs=pl.BlockSpec((n, h1), lambda i: (0, 0)),
        out_shape=jax.ShapeDtypeStruct((n, h1), _F32),
        interpret=INTERPRET,
    )(z, dec_W, dec_b.reshape(1, h1), bn_gamma.reshape(1, h1),
      bn_beta.reshape(1, h1))

    pi, disp, mean_, ret1 = pl.pallas_call(
        _heads_kernel,
        grid=(pl.cdiv(n, _BI_F),),
        in_specs=[
            pl.BlockSpec((_BI_F, h1), lambda i: (i, 0)),
            pl.BlockSpec((3, _BI_F, h2), lambda i: (0, i, 0)),
            pl.BlockSpec((_BI_F, h2), lambda i: (i, 0)),
            pl.BlockSpec((h1, d_in), lambda i: (0, 0)),
            pl.BlockSpec((1, d_in), lambda i: (0, 0)),
            pl.BlockSpec((h1, d_in), lambda i: (0, 0)),
            pl.BlockSpec((1, d_in), lambda i: (0, 0)),
            pl.BlockSpec((h1, d_in), lambda i: (0, 0)),
            pl.BlockSpec((1, d_in), lambda i: (0, 0)),
            pl.BlockSpec((h2, h2), lambda i: (0, 0)),
            pl.BlockSpec((1, 1), lambda i: (0, 0)),
        ],
        out_specs=[
            pl.BlockSpec((_BI_F, d_in), lambda i: (i, 0)),
            pl.BlockSpec((_BI_F, d_in), lambda i: (i, 0)),
            pl.BlockSpec((_BI_F, d_in), lambda i: (i, 0)),
            pl.BlockSpec((_BI_F, 2), lambda i: (i, 0)),
        ],
        out_shape=[
            jax.ShapeDtypeStruct((n, d_in), _F32),
            jax.ShapeDtypeStruct((n, d_in), _F32),
            jax.ShapeDtypeStruct((n, d_in), _F32),
            jax.ShapeDtypeStruct((n, 2), _F32),
        ],
        interpret=INTERPRET,
    )(xr, z, g2, pi_W, pi_b.reshape(1, d_in), disp_W, disp_b.reshape(1, d_in),
      mean_W, mean_b.reshape(1, d_in), bil_W, bil_b.reshape(1, 1))

    return (z[0], z[1], z[2], pi, disp, mean_, rec_adj, ret1)


# speedup vs baseline: 1.8335x; 1.0474x over previous
import jax
import jax.numpy as jnp
from jax.experimental import pallas as pl

INTERPRET = False

_BI_B = 256
_BI_C = 1000
_BI_D = 256
_BI_F = 2000

_BF = jnp.bfloat16
_F32 = jnp.float32


def _bdot(a, b):
    return jax.lax.dot(a.astype(_BF), b.astype(_BF),
                       preferred_element_type=_F32)


def _proj_kernel(xa_ref, xb_ref, xc_ref, w1_ref, p_ref):
    w1 = w1_ref[...].astype(_BF)
    h1 = w1.shape[1]
    for g, x_ref in enumerate((xa_ref, xb_ref, xc_ref)):
        p_ref[:, g * h1:(g + 1) * h1] = jax.lax.dot(
            x_ref[...].astype(_BF), w1,
            preferred_element_type=_F32).astype(_BF)


def _layer1_kernel(adj_ref, p_ref, w2d_ref, q_ref, abf_ref):
    a = adj_ref[...].astype(_BF)
    abf_ref[...] = a
    h = jax.lax.dot(a, p_ref[...], preferred_element_type=_F32)
    h = jnp.maximum(h, 0.0)
    q_ref[...] = jax.lax.dot(h.astype(_BF), w2d_ref[...],
                             preferred_element_type=_F32).astype(_BF)


def _layer2_kernel(adj_ref, q_ref, z_ref, zn_ref, e1_ref):
    a = adj_ref[...]
    zall = jax.lax.dot(a, q_ref[...], preferred_element_type=_F32)
    h2 = zn_ref.shape[-1]
    for g in range(3):
        z_ref[g] = zall[:, g * h2:(g + 1) * h2]
    z0 = zall[:, :h2]
    n = jnp.sqrt(jnp.sum(z0 * z0, axis=1, keepdims=True))
    zn_ref[...] = z0 / jnp.maximum(n, 1e-12)
    e1_ref[...] = jnp.maximum(z0, 0.0)


def _recon_kernel(gn_ref, e1_ref, znb_ref, znf_ref, rec_ref, g2_ref):
    vs = _bdot(gn_ref[...], e1_ref[...])
    nv = jnp.sqrt(jnp.sum(vs * vs, axis=1, keepdims=True))
    g2_ref[...] = jax.nn.sigmoid(vs / jnp.maximum(nv, 1e-12))
    rec = jax.lax.dot_general(
        znb_ref[...].astype(_BF), znf_ref[...].astype(_BF),
        (((1,), (1,)), ((), ())), preferred_element_type=_F32)
    rec_ref[...] = jax.nn.sigmoid(rec)


def _heads_kernel(z_ref, g2_ref, decw_ref, decb_ref, bg_ref, bb2_ref,
                  piw_ref, pib_ref, dw_ref, db_ref,
                  mw_ref, mb_ref, bw_ref, bb_ref,
                  pi_ref, disp_ref, mean_ref, ret_ref):
    xd = _bdot(z_ref[0], decw_ref[...]) + decb_ref[...]
    mu = jnp.mean(xd, axis=0, keepdims=True)
    var = jnp.mean((xd - mu) ** 2, axis=0, keepdims=True)
    xn = (xd - mu) / jnp.sqrt(var + 1e-5) * bg_ref[...] + bb2_ref[...]
    xr = jnp.maximum(xn, 0.0)
    pi_ref[...] = jax.nn.sigmoid(_bdot(xr, piw_ref[...]) + pib_ref[...])
    disp_ref[...] = jnp.clip(
        jax.nn.softplus(_bdot(xr, dw_ref[...]) + db_ref[...]), 1e-4, 1e4)
    mean_ref[...] = jnp.clip(
        jnp.exp(_bdot(xr, mw_ref[...]) + mb_ref[...]), 1e-5, 1e6)
    g2 = g2_ref[...]
    emb1 = jnp.maximum(z_ref[0], 0.0)
    emb3 = jnp.maximum(z_ref[2], 0.0)
    t1 = _bdot(emb1, bw_ref[...])
    t3 = _bdot(emb3, bw_ref[...])
    sc1 = jnp.sum(t1 * g2, axis=1, keepdims=True) + bb_ref[...]
    sc2 = jnp.sum(t3 * g2, axis=1, keepdims=True) + bb_ref[...]
    ret_ref[:, 0:1] = sc1
    ret_ref[:, 1:2] = sc2


def kernel(feat, feat_a, feat_b, adj, graph_neigh, W1, W2, dec_W, dec_b,
           bn_gamma, bn_beta, pi_W, pi_b, disp_W, disp_b, mean_W, mean_b,
           bil_W, bil_b):
    n, d_in = feat.shape
    h1 = W1.shape[1]
    h2 = W2.shape[1]

    zero = jnp.zeros_like(W2)
    w2d = jnp.block([[W2, zero, zero],
                     [zero, W2, zero],
                     [zero, zero, W2]]).astype(_BF)

    p = pl.pallas_call(
        _proj_kernel,
        grid=(1,),
        in_specs=[
            pl.BlockSpec((n, d_in), lambda i: (0, 0)),
            pl.BlockSpec((n, d_in), lambda i: (0, 0)),
            pl.BlockSpec((n, d_in), lambda i: (0, 0)),
            pl.BlockSpec((d_in, h1), lambda i: (0, 0)),
        ],
        out_specs=pl.BlockSpec((n, 3 * h1), lambda i: (0, 0)),
        out_shape=jax.ShapeDtypeStruct((n, 3 * h1), _BF),
        interpret=INTERPRET,
    )(feat, feat_a, feat_b, W1)

    q, adj_bf = pl.pallas_call(
        _layer1_kernel,
        grid=(pl.cdiv(n, _BI_B),),
        in_specs=[
            pl.BlockSpec((_BI_B, n), lambda i: (i, 0)),
            pl.BlockSpec((n, 3 * h1), lambda i: (0, 0)),
            pl.BlockSpec((3 * h1, 3 * h2), lambda i: (0, 0)),
        ],
        out_specs=[
            pl.BlockSpec((_BI_B, 3 * h2), lambda i: (i, 0)),
            pl.BlockSpec((_BI_B, n), lambda i: (i, 0)),
        ],
        out_shape=[
            jax.ShapeDtypeStruct((n, 3 * h2), _BF),
            jax.ShapeDtypeStruct((n, n), _BF),
        ],
        interpret=INTERPRET,
    )(adj, p, w2d)

    z, zn, emb1 = pl.pallas_call(
        _layer2_kernel,
        grid=(pl.cdiv(n, _BI_C),),
        in_specs=[
            pl.BlockSpec((_BI_C, n), lambda i: (i, 0)),
            pl.BlockSpec((n, 3 * h2), lambda i: (0, 0)),
        ],
        out_specs=[
            pl.BlockSpec((3, _BI_C, h2), lambda i: (0, i, 0)),
            pl.BlockSpec((_BI_C, h2), lambda i: (i, 0)),
            pl.BlockSpec((_BI_C, h2), lambda i: (i, 0)),
        ],
        out_shape=[
            jax.ShapeDtypeStruct((3, n, h2), _F32),
            jax.ShapeDtypeStruct((n, h2), _F32),
            jax.ShapeDtypeStruct((n, h2), _F32),
        ],
        interpret=INTERPRET,
    )(adj_bf, q)

    rec_adj, g2 = pl.pallas_call(
        _recon_kernel,
        grid=(pl.cdiv(n, _BI_D),),
        in_specs=[
            pl.BlockSpec((_BI_D, n), lambda i: (i, 0)),
            pl.BlockSpec((n, h2), lambda i: (0, 0)),
            pl.BlockSpec((_BI_D, h2), lambda i: (i, 0)),
            pl.BlockSpec((n, h2), lambda i: (0, 0)),
        ],
        out_specs=[
            pl.BlockSpec((_BI_D, n), lambda i: (i, 0)),
            pl.BlockSpec((_BI_D, h2), lambda i: (i, 0)),
        ],
        out_shape=[
            jax.ShapeDtypeStruct((n, n), _F32),
            jax.ShapeDtypeStruct((n, h2), _F32),
        ],
        interpret=INTERPRET,
    )(graph_neigh, emb1, zn, zn)

    pi, disp, mean_, ret1 = pl.pallas_call(
        _heads_kernel,
        grid=(1,),
        in_specs=[
            pl.BlockSpec((3, n, h2), lambda i: (0, 0, 0)),
            pl.BlockSpec((n, h2), lambda i: (0, 0)),
            pl.BlockSpec((h2, h1), lambda i: (0, 0)),
            pl.BlockSpec((1, h1), lambda i: (0, 0)),
            pl.BlockSpec((1, h1), lambda i: (0, 0)),
            pl.BlockSpec((1, h1), lambda i: (0, 0)),
            pl.BlockSpec((h1, d_in), lambda i: (0, 0)),
            pl.BlockSpec((1, d_in), lambda i: (0, 0)),
            pl.BlockSpec((h1, d_in), lambda i: (0, 0)),
            pl.BlockSpec((1, d_in), lambda i: (0, 0)),
            pl.BlockSpec((h1, d_in), lambda i: (0, 0)),
            pl.BlockSpec((1, d_in), lambda i: (0, 0)),
            pl.BlockSpec((h2, h2), lambda i: (0, 0)),
            pl.BlockSpec((1, 1), lambda i: (0, 0)),
        ],
        out_specs=[
            pl.BlockSpec((n, d_in), lambda i: (0, 0)),
            pl.BlockSpec((n, d_in), lambda i: (0, 0)),
            pl.BlockSpec((n, d_in), lambda i: (0, 0)),
            pl.BlockSpec((n, 2), lambda i: (0, 0)),
        ],
        out_shape=[
            jax.ShapeDtypeStruct((n, d_in), _F32),
            jax.ShapeDtypeStruct((n, d_in), _F32),
            jax.ShapeDtypeStruct((n, d_in), _F32),
            jax.ShapeDtypeStruct((n, 2), _F32),
        ],
        interpret=INTERPRET,
    )(z, g2, dec_W, dec_b.reshape(1, h1), bn_gamma.reshape(1, h1),
      bn_beta.reshape(1, h1), pi_W, pi_b.reshape(1, d_in),
      disp_W, disp_b.reshape(1, d_in), mean_W, mean_b.reshape(1, d_in),
      bil_W, bil_b.reshape(1, 1))

    return (z[0], z[1], z[2], pi, disp, mean_, rec_adj, ret1)
